# Initial kernel scaffold; baseline (speedup 1.0000x reference)
import jax
import jax.numpy as jnp
from jax.experimental import pallas as pl


def kernel(x, router_w, Wd, Wu):
    raise NotImplementedError("write your pallas kernel here")


# speedup vs baseline: 1.7838x; 1.7838x over previous
import functools

import jax
import jax.numpy as jnp
from jax import lax
from jax.experimental import pallas as pl
from jax.experimental.pallas import tpu as pltpu

B, S, H = 2, 2048, 2048
E, TOP_K, R = 8, 2, 64

S_TILE = 512


def _expert_body(idx_ref, gate_ref, x_ref, wd_ref, wu_ref, out_ref):
    k = pl.program_id(2)
    b = pl.program_id(0)
    g = gate_ref[b * TOP_K + k]
    xb = x_ref[0]
    h = jnp.dot(xb, wd_ref[0], preferred_element_type=jnp.float32)
    eo = jnp.dot(h, wu_ref[0], preferred_element_type=jnp.float32)

    @pl.when(k == 0)
    def _():
        out_ref[0] = xb + g * eo

    @pl.when(k != 0)
    def _():
        out_ref[0] = out_ref[0] + g * eo


@jax.jit
def _expert_call(x, wd_t, wu_t, idx_flat, gate_flat):
    grid = (B, S // S_TILE, TOP_K)
    spec = pltpu.PrefetchScalarGridSpec(
        num_scalar_prefetch=2,
        grid=grid,
        in_specs=[
            pl.BlockSpec((1, S_TILE, H), lambda b, s, k, idx, gate: (b, s, 0)),
            pl.BlockSpec((1, H, R),
                         lambda b, s, k, idx, gate: (idx[b * TOP_K + k], 0, 0)),
            pl.BlockSpec((1, R, H),
                         lambda b, s, k, idx, gate: (idx[b * TOP_K + k], 0, 0)),
        ],
        out_specs=pl.BlockSpec((1, S_TILE, H), lambda b, s, k, idx, gate: (b, s, 0)),
    )
    return pl.pallas_call(
        _expert_body,
        grid_spec=spec,
        out_shape=jax.ShapeDtypeStruct((B, S, H), jnp.float32),
        compiler_params=pltpu.CompilerParams(
            dimension_semantics=("parallel", "parallel", "arbitrary"),
        ),
    )(idx_flat, gate_flat, x, wd_t, wu_t)


def kernel(x, router_w, Wd, Wu):
    cls = x[:, 0, :]
    logits = cls @ router_w.T
    topv, topi = lax.top_k(logits, TOP_K)
    gate = jax.nn.softmax(topv, axis=-1)
    idx_flat = topi.reshape(-1).astype(jnp.int32)
    gate_flat = gate.reshape(-1)

    wd_t = Wd.transpose(0, 2, 1)
    wu_t = Wu.transpose(0, 2, 1)
    return _expert_call(x, wd_t, wu_t, idx_flat, gate_flat)


# speedup vs baseline: 1.7985x; 1.0082x over previous
import functools

import jax
import jax.numpy as jnp
from jax import lax
from jax.experimental import pallas as pl
from jax.experimental.pallas import tpu as pltpu

B, S, H = 2, 2048, 2048
E, TOP_K, R = 8, 2, 64

S_TILE = 512


def _expert_body(idx_ref, gate_ref, x_ref, wd_ref, wu_ref, out_ref):
    k = pl.program_id(2)
    b = pl.program_id(0)
    g = gate_ref[b * TOP_K + k]
    xb = x_ref[0]
    h = jnp.dot(xb.astype(jnp.bfloat16), wd_ref[0].astype(jnp.bfloat16),
                preferred_element_type=jnp.float32)
    eo = jnp.dot((g * h).astype(jnp.bfloat16), wu_ref[0].astype(jnp.bfloat16),
                 preferred_element_type=jnp.float32)

    @pl.when(k == 0)
    def _():
        out_ref[0] = xb + eo

    @pl.when(k != 0)
    def _():
        out_ref[0] = out_ref[0] + eo


@jax.jit
def _expert_call(x, wd_t, wu_t, idx_flat, gate_flat):
    grid = (B, S // S_TILE, TOP_K)
    spec = pltpu.PrefetchScalarGridSpec(
        num_scalar_prefetch=2,
        grid=grid,
        in_specs=[
            pl.BlockSpec((1, S_TILE, H), lambda b, s, k, idx, gate: (b, s, 0)),
            pl.BlockSpec((1, H, R),
                         lambda b, s, k, idx, gate: (idx[b * TOP_K + k], 0, 0)),
            pl.BlockSpec((1, R, H),
                         lambda b, s, k, idx, gate: (idx[b * TOP_K + k], 0, 0)),
        ],
        out_specs=pl.BlockSpec((1, S_TILE, H), lambda b, s, k, idx, gate: (b, s, 0)),
    )
    return pl.pallas_call(
        _expert_body,
        grid_spec=spec,
        out_shape=jax.ShapeDtypeStruct((B, S, H), jnp.float32),
        compiler_params=pltpu.CompilerParams(
            dimension_semantics=("parallel", "parallel", "arbitrary"),
        ),
    )(idx_flat, gate_flat, x, wd_t, wu_t)


def kernel(x, router_w, Wd, Wu):
    cls = x[:, 0, :]
    logits = cls @ router_w.T
    topv, topi = lax.top_k(logits, TOP_K)
    gate = jax.nn.softmax(topv, axis=-1)
    idx_flat = topi.reshape(-1).astype(jnp.int32)
    gate_flat = gate.reshape(-1)

    wd_t = Wd.transpose(0, 2, 1)
    wu_t = Wu.transpose(0, 2, 1)
    return _expert_call(x, wd_t, wu_t, idx_flat, gate_flat)


# speedup vs baseline: 2.5515x; 1.4187x over previous
import functools

import jax
import jax.numpy as jnp
from jax import lax
from jax.experimental import pallas as pl
from jax.experimental.pallas import tpu as pltpu

B, S, H = 2, 2048, 2048
E, TOP_K, R = 8, 2, 64
R2 = TOP_K * R

S_TILE = 512


def _expert_body(idx_ref, gate_ref, x_ref, wd0_ref, wd1_ref, wu0_ref, wu1_ref,
                 out_ref, wdc, wuc):
    b = pl.program_id(0)
    s = pl.program_id(1)

    @pl.when(s == 0)
    def _():
        wdc[:, :R] = wd0_ref[0]
        wdc[:, R:] = wd1_ref[0]
        wuc[:R, :] = wu0_ref[0]
        wuc[R:, :] = wu1_ref[0]

    g0 = gate_ref[b * TOP_K]
    g1 = gate_ref[b * TOP_K + 1]
    col = lax.broadcasted_iota(jnp.int32, (1, R2), 1)
    gv = jnp.where(col < R, g0, g1)

    xb = x_ref[0]
    h = jnp.dot(xb.astype(jnp.bfloat16), wdc[...],
                preferred_element_type=jnp.float32)
    hg = (h * gv).astype(jnp.bfloat16)
    eo = jnp.dot(hg, wuc[...],
                 preferred_element_type=jnp.float32)
    out_ref[0] = xb + eo


@jax.jit
def _expert_call(x, wd_t, wu_t, idx_flat, gate_flat):
    grid = (B, S // S_TILE)
    spec = pltpu.PrefetchScalarGridSpec(
        num_scalar_prefetch=2,
        grid=grid,
        in_specs=[
            pl.BlockSpec((1, S_TILE, H), lambda b, s, idx, gate: (b, s, 0)),
            pl.BlockSpec((1, H, R), lambda b, s, idx, gate: (idx[b * TOP_K], 0, 0)),
            pl.BlockSpec((1, H, R), lambda b, s, idx, gate: (idx[b * TOP_K + 1], 0, 0)),
            pl.BlockSpec((1, R, H), lambda b, s, idx, gate: (idx[b * TOP_K], 0, 0)),
            pl.BlockSpec((1, R, H), lambda b, s, idx, gate: (idx[b * TOP_K + 1], 0, 0)),
        ],
        out_specs=pl.BlockSpec((1, S_TILE, H), lambda b, s, idx, gate: (b, s, 0)),
        scratch_shapes=[
            pltpu.VMEM((H, R2), jnp.bfloat16),
            pltpu.VMEM((R2, H), jnp.bfloat16),
        ],
    )
    return pl.pallas_call(
        _expert_body,
        grid_spec=spec,
        out_shape=jax.ShapeDtypeStruct((B, S, H), jnp.float32),
        compiler_params=pltpu.CompilerParams(
            dimension_semantics=("arbitrary", "arbitrary"),
        ),
    )(idx_flat, gate_flat, x, wd_t, wd_t, wu_t, wu_t)


def kernel(x, router_w, Wd, Wu):
    cls = x[:, 0, :]
    logits = cls @ router_w.T
    topv, topi = lax.top_k(logits, TOP_K)
    gate = jax.nn.softmax(topv, axis=-1)
    idx_flat = topi.reshape(-1).astype(jnp.int32)
    gate_flat = gate.reshape(-1)

    wd_t = Wd.transpose(0, 2, 1).astype(jnp.bfloat16)
    wu_t = Wu.transpose(0, 2, 1).astype(jnp.bfloat16)
    return _expert_call(x, wd_t, wu_t, idx_flat, gate_flat)


# speedup vs baseline: 2.6491x; 1.0382x over previous
import functools

import jax
import jax.numpy as jnp
from jax import lax
from jax.experimental import pallas as pl
from jax.experimental.pallas import tpu as pltpu

B, S, H = 2, 2048, 2048
E, TOP_K, R = 8, 2, 64
R2 = TOP_K * R

S_TILE = 512


def _expert_body(idx_ref, gate_ref, x_ref, wd0_ref, wd1_ref, wu0_ref, wu1_ref,
                 out_ref, wdc, wuc):
    b = pl.program_id(0)
    s = pl.program_id(1)

    @pl.when(s == 0)
    def _():
        wdc[:R, :] = wd0_ref[0].astype(jnp.bfloat16)
        wdc[R:, :] = wd1_ref[0].astype(jnp.bfloat16)
        wuc[:, :R] = wu0_ref[0].astype(jnp.bfloat16)
        wuc[:, R:] = wu1_ref[0].astype(jnp.bfloat16)

    g0 = gate_ref[b * TOP_K]
    g1 = gate_ref[b * TOP_K + 1]
    col = lax.broadcasted_iota(jnp.int32, (1, R2), 1)
    gv = jnp.where(col < R, g0, g1)

    nt = (((1,), (1,)), ((), ()))
    xb = x_ref[0]
    h = lax.dot_general(xb.astype(jnp.bfloat16), wdc[...], nt,
                        preferred_element_type=jnp.float32)
    hg = (h * gv).astype(jnp.bfloat16)
    eo = lax.dot_general(hg, wuc[...], nt,
                         preferred_element_type=jnp.float32)
    out_ref[0] = xb + eo


@jax.jit
def _expert_call(x, wd_t, wu_t, idx_flat, gate_flat):
    grid = (B, S // S_TILE)
    spec = pltpu.PrefetchScalarGridSpec(
        num_scalar_prefetch=2,
        grid=grid,
        in_specs=[
            pl.BlockSpec((1, S_TILE, H), lambda b, s, idx, gate: (b, s, 0)),
            pl.BlockSpec((1, R, H), lambda b, s, idx, gate: (idx[b * TOP_K], 0, 0)),
            pl.BlockSpec((1, R, H), lambda b, s, idx, gate: (idx[b * TOP_K + 1], 0, 0)),
            pl.BlockSpec((1, H, R), lambda b, s, idx, gate: (idx[b * TOP_K], 0, 0)),
            pl.BlockSpec((1, H, R), lambda b, s, idx, gate: (idx[b * TOP_K + 1], 0, 0)),
        ],
        out_specs=pl.BlockSpec((1, S_TILE, H), lambda b, s, idx, gate: (b, s, 0)),
        scratch_shapes=[
            pltpu.VMEM((R2, H), jnp.bfloat16),
            pltpu.VMEM((H, R2), jnp.bfloat16),
        ],
    )
    return pl.pallas_call(
        _expert_body,
        grid_spec=spec,
        out_shape=jax.ShapeDtypeStruct((B, S, H), jnp.float32),
        compiler_params=pltpu.CompilerParams(
            dimension_semantics=("arbitrary", "arbitrary"),
        ),
    )(idx_flat, gate_flat, x, wd_t, wd_t, wu_t, wu_t)


def _route_jnp(x, router_w):
    cls = x[:, 0, :]
    logits = cls @ router_w.T
    topv, topi = lax.top_k(logits, TOP_K)
    gate = jax.nn.softmax(topv, axis=-1)
    return topi.reshape(-1).astype(jnp.int32), gate.reshape(-1)


def kernel(x, router_w, Wd, Wu):
    idx_flat, gate_flat = _route_jnp(x, router_w)
    return _expert_call(x, Wd, Wu, idx_flat, gate_flat)
